# baseline (speedup 1.0000x reference)
import functools

import jax
import jax.numpy as jnp
from jax import lax
from jax.experimental import pallas as pl
from jax.experimental.pallas import tpu as pltpu
from jax.experimental.pallas import tpu_sc as plsc

N = 10000
E = 320000
D = 128
NC = 2
NS = 16
NW = NC * NS
CHUNK = 128
EPW = 10240
E_PAD = NW * EPW
NCHUNK = EPW // CHUNK
HCHUNK = NCHUNK // 2
N_PAD = 10240
DUMP_ROW = N
RPS = N_PAD // NS

_sc_mesh = plsc.VectorSubcoreMesh(
    core_axis_name="c", subcore_axis_name="s", num_cores=NC, num_subcores=NS
)


@functools.partial(
    pl.kernel,
    out_type=jax.ShapeDtypeStruct((NC, N_PAD, D), jnp.float32),
    mesh=_sc_mesh,
    scratch_types=[
        pltpu.VMEM((HCHUNK * CHUNK,), jnp.int32),
        pltpu.VMEM((HCHUNK, CHUNK), jnp.int32),
        pltpu.VMEM((2, CHUNK, D), jnp.float32),
        pltpu.VMEM_SHARED((N_PAD, D), jnp.float32),
        pltpu.SemaphoreType.DMA,
        pltpu.SemaphoreType.DMA,
    ],
)
def _propagate(hw_hbm, src_hbm, dst_hbm, zero_hbm, p_hbm,
               src_v, dst_v, rows_v, acc_sh, sem0, sem1):
    c = lax.axis_index("c")
    s = lax.axis_index("s")
    wid = c * NS + s
    pltpu.sync_copy(zero_hbm, acc_sh.at[pl.ds(s * RPS, RPS)])
    plsc.subcore_barrier()

    for half in range(2):
        pltpu.sync_copy(src_hbm.at[wid].at[half], src_v)
        pltpu.sync_copy(dst_hbm.at[wid].at[half], dst_v)

        @pl.loop(0, HCHUNK, step=2)
        def _(j):
            h0 = pltpu.async_copy(
                hw_hbm.at[src_v.at[pl.ds(j * CHUNK, CHUNK)]],
                rows_v.at[0], sem0)
            h1 = pltpu.async_copy(
                hw_hbm.at[src_v.at[pl.ds((j + 1) * CHUNK, CHUNK)]],
                rows_v.at[1], sem1)
            h0.wait()
            pltpu.sync_copy(rows_v.at[0], acc_sh.at[dst_v.at[j]], add=True)
            h1.wait()
            pltpu.sync_copy(rows_v.at[1], acc_sh.at[dst_v.at[j + 1]],
                            add=True)

    plsc.subcore_barrier()
    pltpu.sync_copy(acc_sh.at[pl.ds(s * RPS, RPS)],
                    p_hbm.at[c].at[pl.ds(s * RPS, RPS)])


ROWS_BLK = 1000


def _mm_first_body(x_ref, w_ref, o_ref):
    o_ref[...] = jnp.dot(x_ref[...], w_ref[...],
                         preferred_element_type=jnp.float32)


def _mm_fused_body(p0_ref, p1_ref, b_ref, w_ref, o_ref):
    h = jnp.maximum(p0_ref[...] + p1_ref[...] + b_ref[...], 0.0)
    o_ref[...] = jnp.dot(h, w_ref[...], preferred_element_type=jnp.float32)


def _final_body(p0_ref, p1_ref, b_ref, o_ref):
    o_ref[...] = p0_ref[...] + p1_ref[...] + b_ref[...]


def _mm_first(x, w):
    return pl.pallas_call(
        _mm_first_body,
        grid=(N // ROWS_BLK,),
        in_specs=[
            pl.BlockSpec((ROWS_BLK, D), lambda i: (i, 0)),
            pl.BlockSpec((D, D), lambda i: (0, 0)),
        ],
        out_specs=pl.BlockSpec((ROWS_BLK, D), lambda i: (i, 0)),
        out_shape=jax.ShapeDtypeStruct((N, D), jnp.float32),
    )(x, w)


def _mm_fused(p, b, w):
    return pl.pallas_call(
        _mm_fused_body,
        grid=(N // ROWS_BLK,),
        in_specs=[
            pl.BlockSpec((ROWS_BLK, D), lambda i: (i, 0)),
            pl.BlockSpec((ROWS_BLK, D), lambda i: (i, 0)),
            pl.BlockSpec((1, D), lambda i: (0, 0)),
            pl.BlockSpec((D, D), lambda i: (0, 0)),
        ],
        out_specs=pl.BlockSpec((ROWS_BLK, D), lambda i: (i, 0)),
        out_shape=jax.ShapeDtypeStruct((N, D), jnp.float32),
    )(p[0], p[1], b, w)


def _final(p, b):
    return pl.pallas_call(
        _final_body,
        grid=(N // ROWS_BLK,),
        in_specs=[
            pl.BlockSpec((ROWS_BLK, D), lambda i: (i, 0)),
            pl.BlockSpec((ROWS_BLK, D), lambda i: (i, 0)),
            pl.BlockSpec((1, D), lambda i: (0, 0)),
        ],
        out_specs=pl.BlockSpec((ROWS_BLK, D), lambda i: (i, 0)),
        out_shape=jax.ShapeDtypeStruct((N, D), jnp.float32),
    )(p[0], p[1], b)


def kernel(x, edge_index, W1, b1, W2, b2, W3, b3):
    pad = E_PAD - E
    src = jnp.concatenate(
        [edge_index[0].astype(jnp.int32), jnp.zeros((pad,), jnp.int32)]
    ).reshape(NW, 2, HCHUNK * CHUNK)
    dst = jnp.concatenate(
        [edge_index[1].astype(jnp.int32),
         jnp.full((pad,), DUMP_ROW, jnp.int32)]
    ).reshape(NW, 2, HCHUNK, CHUNK)
    zeros = jnp.zeros((RPS, D), jnp.float32)
    b1r = b1.reshape(1, D)
    b2r = b2.reshape(1, D)
    b3r = b3.reshape(1, D)

    h = _mm_first(x, W1)
    p = _propagate(h, src, dst, zeros)
    h = _mm_fused(p, b1r, W2)
    p = _propagate(h, src, dst, zeros)
    h = _mm_fused(p, b2r, W3)
    p = _propagate(h, src, dst, zeros)
    return _final(p, b3r)


# speedup vs baseline: 1.1751x; 1.1751x over previous
import functools

import jax
import jax.numpy as jnp
from jax import lax
from jax.experimental import pallas as pl
from jax.experimental.pallas import tpu as pltpu
from jax.experimental.pallas import tpu_sc as plsc

N = 10000
E = 320000
D = 128
NC = 2
NS = 16
NW = NC * NS
CHUNK = 128
EPW = 10240
E_PAD = NW * EPW
NCHUNK = EPW // CHUNK
HCHUNK = NCHUNK // 2
N_PAD = 10240
DUMP_ROW = N
RPS = N_PAD // NS

_sc_mesh = plsc.VectorSubcoreMesh(
    core_axis_name="c", subcore_axis_name="s", num_cores=NC, num_subcores=NS
)


@functools.partial(
    pl.kernel,
    out_type=jax.ShapeDtypeStruct((NC, N_PAD, D), jnp.float32),
    mesh=_sc_mesh,
    scratch_types=[
        pltpu.VMEM((HCHUNK * CHUNK,), jnp.int32),
        pltpu.VMEM((HCHUNK, CHUNK), jnp.int32),
        pltpu.VMEM((2, CHUNK, D), jnp.float32),
        pltpu.VMEM_SHARED((N_PAD, D), jnp.float32),
        pltpu.SemaphoreType.DMA,
        pltpu.SemaphoreType.DMA,
    ],
)
def _propagate(hw_hbm, src_hbm, dst_hbm, zero_hbm, p_hbm,
               src_v, dst_v, rows_v, acc_sh, sem0, sem1):
    c = lax.axis_index("c")
    s = lax.axis_index("s")
    wid = c * NS + s
    pltpu.sync_copy(zero_hbm, acc_sh.at[pl.ds(s * RPS, RPS)])
    plsc.subcore_barrier()

    for half in range(2):
        pltpu.sync_copy(src_hbm.at[wid].at[half], src_v)
        pltpu.sync_copy(dst_hbm.at[wid].at[half], dst_v)

        @pl.loop(0, HCHUNK, step=2)
        def _(j):
            h0 = pltpu.async_copy(
                hw_hbm.at[src_v.at[pl.ds(j * CHUNK, CHUNK)]],
                rows_v.at[0], sem0)
            h1 = pltpu.async_copy(
                hw_hbm.at[src_v.at[pl.ds((j + 1) * CHUNK, CHUNK)]],
                rows_v.at[1], sem1)
            h0.wait()
            pltpu.sync_copy(rows_v.at[0], acc_sh.at[dst_v.at[j]], add=True)
            h1.wait()
            pltpu.sync_copy(rows_v.at[1], acc_sh.at[dst_v.at[j + 1]],
                            add=True)

    plsc.subcore_barrier()
    pltpu.sync_copy(acc_sh.at[pl.ds(s * RPS, RPS)],
                    p_hbm.at[c].at[pl.ds(s * RPS, RPS)])


ROWS_BLK = 1000


def _mm_first_body(x_ref, w_ref, o_ref):
    o_ref[...] = jnp.dot(x_ref[...], w_ref[...],
                         preferred_element_type=jnp.float32)


def _mm_fused_body(p0_ref, p1_ref, b_ref, w_ref, o_ref):
    h = jnp.maximum(p0_ref[...] + p1_ref[...] + b_ref[...], 0.0)
    o_ref[...] = jnp.dot(h, w_ref[...], preferred_element_type=jnp.float32)


def _final_body(p0_ref, p1_ref, b_ref, o_ref):
    o_ref[...] = p0_ref[...] + p1_ref[...] + b_ref[...]


def _mm_first(x, w):
    return pl.pallas_call(
        _mm_first_body,
        grid=(N // ROWS_BLK,),
        in_specs=[
            pl.BlockSpec((ROWS_BLK, D), lambda i: (i, 0)),
            pl.BlockSpec((D, D), lambda i: (0, 0)),
        ],
        out_specs=pl.BlockSpec((ROWS_BLK, D), lambda i: (i, 0)),
        out_shape=jax.ShapeDtypeStruct((N, D), jnp.float32),
    )(x, w)


def _mm_fused(p, b, w):
    return pl.pallas_call(
        _mm_fused_body,
        grid=(N // ROWS_BLK,),
        in_specs=[
            pl.BlockSpec((ROWS_BLK, D), lambda i: (i, 0)),
            pl.BlockSpec((ROWS_BLK, D), lambda i: (i, 0)),
            pl.BlockSpec((1, D), lambda i: (0, 0)),
            pl.BlockSpec((D, D), lambda i: (0, 0)),
        ],
        out_specs=pl.BlockSpec((ROWS_BLK, D), lambda i: (i, 0)),
        out_shape=jax.ShapeDtypeStruct((N, D), jnp.float32),
    )(p[0], p[1], b, w)


def _final(p, b):
    return pl.pallas_call(
        _final_body,
        grid=(N // ROWS_BLK,),
        in_specs=[
            pl.BlockSpec((ROWS_BLK, D), lambda i: (i, 0)),
            pl.BlockSpec((ROWS_BLK, D), lambda i: (i, 0)),
            pl.BlockSpec((1, D), lambda i: (0, 0)),
        ],
        out_specs=pl.BlockSpec((ROWS_BLK, D), lambda i: (i, 0)),
        out_shape=jax.ShapeDtypeStruct((N, D), jnp.float32),
    )(p[0], p[1], b)


def kernel(x, edge_index, W1, b1, W2, b2, W3, b3):
    ppw = EPW - E // NW
    src = jnp.concatenate(
        [edge_index[0].astype(jnp.int32).reshape(NW, E // NW),
         jnp.zeros((NW, ppw), jnp.int32)], axis=1
    ).reshape(NW, 2, HCHUNK * CHUNK)
    dst = jnp.concatenate(
        [edge_index[1].astype(jnp.int32).reshape(NW, E // NW),
         jnp.broadcast_to(N + jnp.arange(ppw, dtype=jnp.int32),
                          (NW, ppw))], axis=1
    ).reshape(NW, 2, HCHUNK, CHUNK)
    zeros = jnp.zeros((RPS, D), jnp.float32)
    b1r = b1.reshape(1, D)
    b2r = b2.reshape(1, D)
    b3r = b3.reshape(1, D)

    h = _mm_first(x, W1)
    p = _propagate(h, src, dst, zeros)
    h = _mm_fused(p, b1r, W2)
    p = _propagate(h, src, dst, zeros)
    h = _mm_fused(p, b2r, W3)
    p = _propagate(h, src, dst, zeros)
    return _final(p, b3r)


# speedup vs baseline: 2.8497x; 2.4250x over previous
import functools

import jax
import jax.numpy as jnp
from jax import lax
from jax.experimental import pallas as pl
from jax.experimental.pallas import tpu as pltpu
from jax.experimental.pallas import tpu_sc as plsc

N = 10000
E = 320000
D = 128
NC = 2
NS = 16
NW = NC * NS
EPW = E // NW
CHUNK = 80
NCHUNK = EPW // CHUNK
N_PAD = 10240
RPS = N_PAD // NS

_sc_mesh = plsc.VectorSubcoreMesh(
    core_axis_name="c", subcore_axis_name="s", num_cores=NC, num_subcores=NS
)


@functools.partial(
    pl.kernel,
    out_type=jax.ShapeDtypeStruct((NC, N_PAD, D), jnp.float32),
    mesh=_sc_mesh,
    scratch_types=[
        pltpu.VMEM((EPW,), jnp.int32),
        pltpu.VMEM((NCHUNK, CHUNK), jnp.int32),
        pltpu.VMEM((2, CHUNK, D), jnp.float32),
        pltpu.VMEM_SHARED((N_PAD, D), jnp.float32),
        pltpu.SemaphoreType.DMA,
        pltpu.SemaphoreType.DMA,
    ],
)
def _propagate(hw_hbm, src_hbm, dst_hbm, zero_hbm, p_hbm,
               src_v, dst_v, rows_v, acc_sh, sem0, sem1):
    c = lax.axis_index("c")
    s = lax.axis_index("s")
    wid = c * NS + s
    pltpu.sync_copy(src_hbm.at[wid], src_v)
    pltpu.sync_copy(dst_hbm.at[wid], dst_v)
    pltpu.sync_copy(zero_hbm, acc_sh.at[pl.ds(s * RPS, RPS)])
    plsc.subcore_barrier()

    @pl.loop(0, NCHUNK - 1, step=2)
    def _(j):
        h0 = pltpu.async_copy(
            hw_hbm.at[src_v.at[pl.ds(j * CHUNK, CHUNK)]], rows_v.at[0],
            sem0)
        h1 = pltpu.async_copy(
            hw_hbm.at[src_v.at[pl.ds((j + 1) * CHUNK, CHUNK)]],
            rows_v.at[1], sem1)
        h0.wait()
        pltpu.sync_copy(rows_v.at[0], acc_sh.at[dst_v.at[j]], add=True)
        h1.wait()
        pltpu.sync_copy(rows_v.at[1], acc_sh.at[dst_v.at[j + 1]], add=True)

    hl = pltpu.async_copy(
        hw_hbm.at[src_v.at[pl.ds((NCHUNK - 1) * CHUNK, CHUNK)]],
        rows_v.at[0], sem0)
    hl.wait()
    pltpu.sync_copy(rows_v.at[0], acc_sh.at[dst_v.at[NCHUNK - 1]], add=True)

    plsc.subcore_barrier()
    pltpu.sync_copy(acc_sh.at[pl.ds(s * RPS, RPS)],
                    p_hbm.at[c].at[pl.ds(s * RPS, RPS)])


ROWS_BLK = 1000


def _mm_first_body(x_ref, w_ref, o_ref):
    o_ref[...] = jnp.dot(x_ref[...], w_ref[...],
                         preferred_element_type=jnp.float32)


def _mm_fused_body(p0_ref, p1_ref, b_ref, w_ref, o_ref):
    h = jnp.maximum(p0_ref[...] + p1_ref[...] + b_ref[...], 0.0)
    o_ref[...] = jnp.dot(h, w_ref[...], preferred_element_type=jnp.float32)


def _final_body(p0_ref, p1_ref, b_ref, o_ref):
    o_ref[...] = p0_ref[...] + p1_ref[...] + b_ref[...]


def _mm_first(x, w):
    return pl.pallas_call(
        _mm_first_body,
        grid=(N // ROWS_BLK,),
        in_specs=[
            pl.BlockSpec((ROWS_BLK, D), lambda i: (i, 0)),
            pl.BlockSpec((D, D), lambda i: (0, 0)),
        ],
        out_specs=pl.BlockSpec((ROWS_BLK, D), lambda i: (i, 0)),
        out_shape=jax.ShapeDtypeStruct((N, D), jnp.float32),
    )(x, w)


def _mm_fused(p, b, w):
    return pl.pallas_call(
        _mm_fused_body,
        grid=(N // ROWS_BLK,),
        in_specs=[
            pl.BlockSpec((ROWS_BLK, D), lambda i: (i, 0)),
            pl.BlockSpec((ROWS_BLK, D), lambda i: (i, 0)),
            pl.BlockSpec((1, D), lambda i: (0, 0)),
            pl.BlockSpec((D, D), lambda i: (0, 0)),
        ],
        out_specs=pl.BlockSpec((ROWS_BLK, D), lambda i: (i, 0)),
        out_shape=jax.ShapeDtypeStruct((N, D), jnp.float32),
    )(p[0], p[1], b, w)


def _final(p, b):
    return pl.pallas_call(
        _final_body,
        grid=(N // ROWS_BLK,),
        in_specs=[
            pl.BlockSpec((ROWS_BLK, D), lambda i: (i, 0)),
            pl.BlockSpec((ROWS_BLK, D), lambda i: (i, 0)),
            pl.BlockSpec((1, D), lambda i: (0, 0)),
        ],
        out_specs=pl.BlockSpec((ROWS_BLK, D), lambda i: (i, 0)),
        out_shape=jax.ShapeDtypeStruct((N, D), jnp.float32),
    )(p[0], p[1], b)


def kernel(x, edge_index, W1, b1, W2, b2, W3, b3):
    src = edge_index[0].astype(jnp.int32).reshape(NW, EPW)
    dst = edge_index[1].astype(jnp.int32).reshape(NW, NCHUNK, CHUNK)
    zeros = jnp.zeros((RPS, D), jnp.float32)
    b1r = b1.reshape(1, D)
    b2r = b2.reshape(1, D)
    b3r = b3.reshape(1, D)

    h = _mm_first(x, W1)
    p = _propagate(h, src, dst, zeros)
    h = _mm_fused(p, b1r, W2)
    p = _propagate(h, src, dst, zeros)
    h = _mm_fused(p, b2r, W3)
    p = _propagate(h, src, dst, zeros)
    return _final(p, b3r)


# speedup vs baseline: 3.6516x; 1.2814x over previous
import functools

import jax
import jax.numpy as jnp
from jax import lax
from jax.experimental import pallas as pl
from jax.experimental.pallas import tpu as pltpu
from jax.experimental.pallas import tpu_sc as plsc

N = 10000
E = 320000
D = 128
NC = 2
NS = 16
NW = NC * NS
EPW = E // NW
CHUNK = 80
NCHUNK = EPW // CHUNK
N_PAD = 10240
RPS = N_PAD // NS

_sc_mesh = plsc.VectorSubcoreMesh(
    core_axis_name="c", subcore_axis_name="s", num_cores=NC, num_subcores=NS
)


@functools.partial(
    pl.kernel,
    out_type=jax.ShapeDtypeStruct((NC, N_PAD, D), jnp.float32),
    mesh=_sc_mesh,
    scratch_types=[
        pltpu.VMEM((EPW,), jnp.int32),
        pltpu.VMEM((NCHUNK, CHUNK), jnp.int32),
        pltpu.VMEM((2, CHUNK, D), jnp.float32),
        pltpu.VMEM_SHARED((N_PAD, D), jnp.float32),
        pltpu.SemaphoreType.DMA,
        pltpu.SemaphoreType.DMA,
    ],
)
def _propagate(hw_hbm, src_hbm, dst_hbm, zero_hbm, p_hbm,
               src_v, dst_v, rows_v, acc_sh, sem0, sem1):
    c = lax.axis_index("c")
    s = lax.axis_index("s")
    wid = c * NS + s
    pltpu.sync_copy(src_hbm.at[wid], src_v)
    pltpu.sync_copy(dst_hbm.at[wid], dst_v)
    pltpu.sync_copy(zero_hbm, acc_sh.at[pl.ds(s * RPS, RPS)])
    plsc.subcore_barrier()

    def _issue(j, b, sem):
        return pltpu.async_copy(
            hw_hbm.at[src_v.at[pl.ds(j * CHUNK, CHUNK)]], rows_v.at[b],
            sem)

    def _wait(b, sem):
        pltpu.make_async_copy(hw_hbm.at[src_v.at[pl.ds(0, CHUNK)]],
                              rows_v.at[b], sem).wait()

    def _scat(j, b):
        pltpu.sync_copy(rows_v.at[b], acc_sh.at[dst_v.at[j]], add=True)

    _issue(0, 0, sem0)
    _issue(1, 1, sem1)

    @pl.loop(0, NCHUNK - 5, step=4)
    def _(j):
        _wait(0, sem0)
        _scat(j, 0)
        h2 = _issue(j + 2, 0, sem0)
        _wait(1, sem1)
        _scat(j + 1, 1)
        h3 = _issue(j + 3, 1, sem1)
        h2.wait()
        _scat(j + 2, 0)
        _issue(j + 4, 0, sem0)
        h3.wait()
        _scat(j + 3, 1)
        _issue(j + 5, 1, sem1)

    t = NCHUNK - 5
    _wait(0, sem0)
    _scat(t, 0)
    _issue(t + 2, 0, sem0)
    _wait(1, sem1)
    _scat(t + 1, 1)
    _issue(t + 3, 1, sem1)
    _wait(0, sem0)
    _scat(t + 2, 0)
    _issue(t + 4, 0, sem0)
    _wait(1, sem1)
    _scat(t + 3, 1)
    _wait(0, sem0)
    _scat(t + 4, 0)

    plsc.subcore_barrier()
    pltpu.sync_copy(acc_sh.at[pl.ds(s * RPS, RPS)],
                    p_hbm.at[c].at[pl.ds(s * RPS, RPS)])


ROWS_BLK = 1000


def _mm_first_body(x_ref, w_ref, o_ref):
    o_ref[...] = jnp.dot(x_ref[...], w_ref[...],
                         preferred_element_type=jnp.float32)


def _mm_fused_body(p0_ref, p1_ref, b_ref, w_ref, o_ref):
    h = jnp.maximum(p0_ref[...] + p1_ref[...] + b_ref[...], 0.0)
    o_ref[...] = jnp.dot(h, w_ref[...], preferred_element_type=jnp.float32)


def _final_body(p0_ref, p1_ref, b_ref, o_ref):
    o_ref[...] = p0_ref[...] + p1_ref[...] + b_ref[...]


def _mm_first(x, w):
    return pl.pallas_call(
        _mm_first_body,
        grid=(N // ROWS_BLK,),
        in_specs=[
            pl.BlockSpec((ROWS_BLK, D), lambda i: (i, 0)),
            pl.BlockSpec((D, D), lambda i: (0, 0)),
        ],
        out_specs=pl.BlockSpec((ROWS_BLK, D), lambda i: (i, 0)),
        out_shape=jax.ShapeDtypeStruct((N, D), jnp.float32),
    )(x, w)


def _mm_fused(p, b, w):
    return pl.pallas_call(
        _mm_fused_body,
        grid=(N // ROWS_BLK,),
        in_specs=[
            pl.BlockSpec((ROWS_BLK, D), lambda i: (i, 0)),
            pl.BlockSpec((ROWS_BLK, D), lambda i: (i, 0)),
            pl.BlockSpec((1, D), lambda i: (0, 0)),
            pl.BlockSpec((D, D), lambda i: (0, 0)),
        ],
        out_specs=pl.BlockSpec((ROWS_BLK, D), lambda i: (i, 0)),
        out_shape=jax.ShapeDtypeStruct((N, D), jnp.float32),
    )(p[0], p[1], b, w)


def _final(p, b):
    return pl.pallas_call(
        _final_body,
        grid=(N // ROWS_BLK,),
        in_specs=[
            pl.BlockSpec((ROWS_BLK, D), lambda i: (i, 0)),
            pl.BlockSpec((ROWS_BLK, D), lambda i: (i, 0)),
            pl.BlockSpec((1, D), lambda i: (0, 0)),
        ],
        out_specs=pl.BlockSpec((ROWS_BLK, D), lambda i: (i, 0)),
        out_shape=jax.ShapeDtypeStruct((N, D), jnp.float32),
    )(p[0], p[1], b)


def kernel(x, edge_index, W1, b1, W2, b2, W3, b3):
    src = edge_index[0].astype(jnp.int32).reshape(NW, EPW)
    dst = edge_index[1].astype(jnp.int32).reshape(NW, NCHUNK, CHUNK)
    zeros = jnp.zeros((RPS, D), jnp.float32)
    b1r = b1.reshape(1, D)
    b2r = b2.reshape(1, D)
    b3r = b3.reshape(1, D)

    h = _mm_first(x, W1)
    p = _propagate(h, src, dst, zeros)
    h = _mm_fused(p, b1r, W2)
    p = _propagate(h, src, dst, zeros)
    h = _mm_fused(p, b2r, W3)
    p = _propagate(h, src, dst, zeros)
    return _final(p, b3r)
